# baseline (speedup 1.0000x reference)
import functools

import jax
import jax.numpy as jnp
from jax import lax
from jax.experimental import pallas as pl
from jax.experimental.pallas import tpu as pltpu
from jax.experimental.pallas import tpu_sc as plsc

NC = 2
NS = 16
L = 16
NW = NC * NS

B = 16384
D = 32
BPW = B // NW
CHUNK = 128
NCHUNK = BPW // CHUNK


def _bpr_body(user_h, pos_h, neg_h, ut_h, it_h, pos_out_h, neg_out_h,
              uidx, pidx, nidx, urows, prows, nrows, outp, outn, sem):
    wid = lax.axis_index("s") * NC + lax.axis_index("c")
    base = wid * BPW

    for j in range(NCHUNK):
        off = base + j * CHUNK
        pltpu.sync_copy(user_h.at[pl.ds(off, CHUNK)], uidx.at[j])
        pltpu.sync_copy(pos_h.at[pl.ds(off, CHUNK)], pidx.at[j])
        pltpu.sync_copy(neg_h.at[pl.ds(off, CHUNK)], nidx.at[j])

    descs = []
    for j in range(NCHUNK):
        dst = pl.ds(j * CHUNK, CHUNK)
        descs.append(pltpu.async_copy(ut_h.at[uidx.at[j]], urows.at[dst], sem))
        descs.append(pltpu.async_copy(it_h.at[pidx.at[j]], prows.at[dst], sem))
        descs.append(pltpu.async_copy(it_h.at[nidx.at[j]], nrows.at[dst], sem))
    for d in descs:
        d.wait()

    def group_body(g, carry):
        row0 = pl.multiple_of(g * L, L)
        rows = row0 + lax.iota(jnp.int32, L)
        accp = jnp.zeros((L,), jnp.float32)
        accn = jnp.zeros((L,), jnp.float32)
        for f in range(D):
            col = jnp.full((L,), f, jnp.int32)
            uv = plsc.load_gather(urows, [rows, col])
            pv = plsc.load_gather(prows, [rows, col])
            nv = plsc.load_gather(nrows, [rows, col])
            accp = accp + uv * pv
            accn = accn + uv * nv
        outp[pl.ds(row0, L)] = accp
        outn[pl.ds(row0, L)] = accn
        return carry

    lax.fori_loop(0, BPW // L, group_body, 0)

    pltpu.sync_copy(outp, pos_out_h.at[pl.ds(base, BPW)])
    pltpu.sync_copy(outn, neg_out_h.at[pl.ds(base, BPW)])


@jax.jit
def _bpr(user, pos_item, neg_item, user_table, item_table):
    run = pl.kernel(
        _bpr_body,
        out_type=(jax.ShapeDtypeStruct((B,), jnp.float32),
                  jax.ShapeDtypeStruct((B,), jnp.float32)),
        mesh=plsc.VectorSubcoreMesh(core_axis_name="c", subcore_axis_name="s"),
        scratch_types=[
            pltpu.VMEM((NCHUNK, CHUNK), jnp.int32),
            pltpu.VMEM((NCHUNK, CHUNK), jnp.int32),
            pltpu.VMEM((NCHUNK, CHUNK), jnp.int32),
            pltpu.VMEM((BPW, D), jnp.float32),
            pltpu.VMEM((BPW, D), jnp.float32),
            pltpu.VMEM((BPW, D), jnp.float32),
            pltpu.VMEM((BPW,), jnp.float32),
            pltpu.VMEM((BPW,), jnp.float32),
            pltpu.SemaphoreType.DMA,
        ],
        compiler_params=pltpu.CompilerParams(needs_layout_passes=False, use_tc_tiling_on_sc=False),
    )
    return run(user, pos_item, neg_item, user_table, item_table)


def kernel(user, pos_item, neg_item, user_table, item_table):
    return _bpr(user, pos_item, neg_item, user_table, item_table)


# speedup vs baseline: 3.2434x; 3.2434x over previous
import jax
import jax.numpy as jnp
from jax import lax
from jax.experimental import pallas as pl
from jax.experimental.pallas import tpu as pltpu
from jax.experimental.pallas import tpu_sc as plsc

NC = 2
NS = 16
L = 16
NW = NC * NS

B = 16384
D = 32
BPW = B // NW
NBUF = 4
LANE = 128
NGRP = BPW // L


def _bpr_body(user_h, pos_h, neg_h, ut_h, it_h, pos_out_h, neg_out_h,
              uidx, pidx, nidx, ubuf, pbuf, nbuf, outp, outn, sem):
    wid = lax.axis_index("s") * NC + lax.axis_index("c")
    base = wid * BPW

    du = pltpu.async_copy(user_h.at[pl.ds(base, BPW)], uidx, sem)
    dp = pltpu.async_copy(pos_h.at[pl.ds(base, BPW)], pidx, sem)
    dn = pltpu.async_copy(neg_h.at[pl.ds(base, BPW)], nidx, sem)
    du.wait()
    dp.wait()
    dn.wait()

    c_lo = lax.iota(jnp.int32, L)
    c_hi = c_lo + L

    def fire(ru, rp, rn, slot):
        ou = pl.multiple_of((ru >> 7) * LANE, LANE)
        op = pl.multiple_of((rp >> 7) * LANE, LANE)
        on = pl.multiple_of((rn >> 7) * LANE, LANE)
        pltpu.async_copy(ut_h.at[:, pl.ds(ou, LANE)], ubuf.at[slot], sem)
        pltpu.async_copy(it_h.at[:, pl.ds(op, LANE)], pbuf.at[slot], sem)
        pltpu.async_copy(it_h.at[:, pl.ds(on, LANE)], nbuf.at[slot], sem)

    def drain():
        pltpu.make_async_copy(ut_h.at[:, pl.ds(0, LANE)], ubuf.at[0], sem).wait()
        pltpu.make_async_copy(it_h.at[:, pl.ds(0, LANE)], pbuf.at[0], sem).wait()
        pltpu.make_async_copy(it_h.at[:, pl.ds(0, LANE)], nbuf.at[0], sem).wait()

    vu0 = uidx[pl.ds(0, L)]
    vp0 = pidx[pl.ds(0, L)]
    vn0 = nidx[pl.ds(0, L)]
    for k in range(NBUF):
        fire(vu0[k], vp0[k], vn0[k], k)

    def group(g, carry):
        accp, accn = carry
        gb = pl.multiple_of(g * L, L)
        vu = uidx[pl.ds(gb, L)]
        vp = pidx[pl.ds(gb, L)]
        vn = nidx[pl.ds(gb, L)]
        nb = pl.multiple_of(jnp.minimum(gb + L, BPW - L), L)
        nvu = uidx[pl.ds(nb, L)]
        nvp = pidx[pl.ds(nb, L)]
        nvn = nidx[pl.ds(nb, L)]
        for k in range(L):
            i = gb + k
            drain()
            slot = jnp.full((L,), k % NBUF, jnp.int32)
            lu = jnp.full((L,), 0, jnp.int32) + (vu[k] & 127)
            lp = jnp.full((L,), 0, jnp.int32) + (vp[k] & 127)
            ln = jnp.full((L,), 0, jnp.int32) + (vn[k] & 127)
            u0 = plsc.load_gather(ubuf, [slot, c_lo, lu])
            u1 = plsc.load_gather(ubuf, [slot, c_hi, lu])
            p0 = plsc.load_gather(pbuf, [slot, c_lo, lp])
            p1 = plsc.load_gather(pbuf, [slot, c_hi, lp])
            n0 = plsc.load_gather(nbuf, [slot, c_lo, ln])
            n1 = plsc.load_gather(nbuf, [slot, c_hi, ln])
            sp = lax.reduce_sum_p.bind(u0 * p0 + u1 * p1, axes=(0,))
            sn = lax.reduce_sum_p.bind(u0 * n0 + u1 * n1, axes=(0,))
            accp = jnp.where(c_lo == k, sp, accp)
            accn = jnp.where(c_lo == k, sn, accn)
            if k == L - 1:
                outp[pl.ds(gb, L)] = accp
                outn[pl.ds(gb, L)] = accn
                accp = jnp.zeros((L,), jnp.float32)
                accn = jnp.zeros((L,), jnp.float32)
            if k + NBUF < L:
                ru, rp, rn = vu[k + NBUF], vp[k + NBUF], vn[k + NBUF]
            else:
                ru, rp, rn = nvu[k + NBUF - L], nvp[k + NBUF - L], nvn[k + NBUF - L]

            @pl.when(i + NBUF < BPW)
            def _():
                fire(ru, rp, rn, k % NBUF)

        return accp, accn

    zero = jnp.zeros((L,), jnp.float32)
    lax.fori_loop(0, NGRP, group, (zero, zero))

    pltpu.sync_copy(outp, pos_out_h.at[pl.ds(base, BPW)])
    pltpu.sync_copy(outn, neg_out_h.at[pl.ds(base, BPW)])


@jax.jit
def _bpr(user, pos_item, neg_item, user_table, item_table):
    run = pl.kernel(
        _bpr_body,
        out_type=(jax.ShapeDtypeStruct((B,), jnp.float32),
                  jax.ShapeDtypeStruct((B,), jnp.float32)),
        mesh=plsc.VectorSubcoreMesh(core_axis_name="c", subcore_axis_name="s"),
        scratch_types=[
            pltpu.VMEM((BPW,), jnp.int32),
            pltpu.VMEM((BPW,), jnp.int32),
            pltpu.VMEM((BPW,), jnp.int32),
            pltpu.VMEM((NBUF, D, LANE), jnp.float32),
            pltpu.VMEM((NBUF, D, LANE), jnp.float32),
            pltpu.VMEM((NBUF, D, LANE), jnp.float32),
            pltpu.VMEM((BPW,), jnp.float32),
            pltpu.VMEM((BPW,), jnp.float32),
            pltpu.SemaphoreType.DMA,
        ],
        compiler_params=pltpu.CompilerParams(needs_layout_passes=False),
    )
    return run(user, pos_item, neg_item, user_table.T, item_table.T)


def kernel(user, pos_item, neg_item, user_table, item_table):
    return _bpr(user, pos_item, neg_item, user_table, item_table)
